# Initial kernel scaffold; baseline (speedup 1.0000x reference)
import jax
import jax.numpy as jnp
from jax.experimental import pallas as pl


def kernel(x, edge_index, W1, b1, W2, b2, W3, b3, W4, b4):
    raise NotImplementedError("write your pallas kernel here")


# speedup vs baseline: 12.9872x; 12.9872x over previous
import functools

import jax
import jax.numpy as jnp
from jax import lax
from jax.experimental import pallas as pl
from jax.experimental.pallas import tpu as pltpu
from jax.experimental.pallas import tpu_sc as plsc

_N = 10000
_D = 128
_HID = 128
_NCLS = 16
_E = 320000

_NTILES = 16
_NCORES = 2
_CHUNK = 128

_NPAD = 10240
_EPAD = 323584
_NODES_PER_TILE = _NPAD // _NTILES
_EDGES_FULL_PER_TILE = _EPAD // _NTILES
_CHUNKS_FULL = _EDGES_FULL_PER_TILE // _CHUNK
_EDGES_HALF_PER_TILE = _EPAD // (_NTILES * _NCORES)
_CHUNKS_HALF = _EDGES_HALF_PER_TILE // _CHUNK

_VL = 16


def _fill(ref, n, value):
    def body(i, _):
        ref[pl.ds(i * _VL, _VL)] = jnp.full((_VL,), value, jnp.float32)
        return 0
    lax.fori_loop(0, n // _VL, body, 0)


def _deg_body(edges_hbm, deg_out, idx_v, ones_v, zeros_v, deg_sh):
    c = lax.axis_index("c")
    s = lax.axis_index("s")
    _fill(ones_v, _CHUNK, 1.0)
    _fill(zeros_v, _NODES_PER_TILE, 0.0)
    off = s * _NODES_PER_TILE
    pltpu.sync_copy(zeros_v, deg_sh.at[pl.ds(off, _NODES_PER_TILE)])
    plsc.subcore_barrier()

    base = s * _EDGES_FULL_PER_TILE

    def body(j, _):
        pltpu.sync_copy(edges_hbm.at[c, pl.ds(base + j * _CHUNK, _CHUNK)], idx_v)
        pltpu.sync_copy(ones_v, deg_sh.at[idx_v], add=True)
        return 0

    lax.fori_loop(0, _CHUNKS_FULL, body, 0)
    plsc.subcore_barrier()
    pltpu.sync_copy(deg_sh.at[pl.ds(off, _NODES_PER_TILE)],
                    deg_out.at[c, pl.ds(off, _NODES_PER_TILE)])


_deg_call = functools.partial(
    pl.kernel,
    out_type=jax.ShapeDtypeStruct((_NCORES, _NPAD), jnp.float32),
    mesh=plsc.VectorSubcoreMesh(core_axis_name="c", subcore_axis_name="s"),
    scratch_types=[
        pltpu.VMEM((_CHUNK,), jnp.int32),
        pltpu.VMEM((_CHUNK,), jnp.float32),
        pltpu.VMEM((_NODES_PER_TILE,), jnp.float32),
        pltpu.VMEM_SHARED((_NPAD,), jnp.float32),
    ],
)(_deg_body)


def _seg_body(edges_hbm, t_hbm, r_out, idx_v, idx2_v, val_v, nb2_v, t_sh, r_sh):
    c = lax.axis_index("c")
    s = lax.axis_index("s")
    off = s * _NODES_PER_TILE
    pltpu.sync_copy(t_hbm.at[pl.ds(off, _NODES_PER_TILE)],
                    t_sh.at[pl.ds(off, _NODES_PER_TILE)])
    _fill(nb2_v, _NODES_PER_TILE, 0.0)
    pltpu.sync_copy(nb2_v, r_sh.at[pl.ds(off, _NODES_PER_TILE)])
    plsc.subcore_barrier()

    base = (c * _NTILES + s) * _EDGES_HALF_PER_TILE

    def ebody(j, _):
        pltpu.sync_copy(edges_hbm.at[0, pl.ds(base + j * _CHUNK, _CHUNK)], idx_v)
        pltpu.sync_copy(edges_hbm.at[1, pl.ds(base + j * _CHUNK, _CHUNK)], idx2_v)
        pltpu.sync_copy(t_sh.at[idx_v], val_v)
        pltpu.sync_copy(val_v, r_sh.at[idx2_v], add=True)
        return 0

    lax.fori_loop(0, _CHUNKS_HALF, ebody, 0)
    plsc.subcore_barrier()
    pltpu.sync_copy(r_sh.at[pl.ds(off, _NODES_PER_TILE)],
                    r_out.at[c, pl.ds(off, _NODES_PER_TILE)])


_seg_call = functools.partial(
    pl.kernel,
    out_type=jax.ShapeDtypeStruct((_NCORES, _NPAD), jnp.float32),
    mesh=plsc.VectorSubcoreMesh(core_axis_name="c", subcore_axis_name="s"),
    scratch_types=[
        pltpu.VMEM((_CHUNK,), jnp.int32),
        pltpu.VMEM((_CHUNK,), jnp.int32),
        pltpu.VMEM((_CHUNK,), jnp.float32),
        pltpu.VMEM((_NODES_PER_TILE,), jnp.float32),
        pltpu.VMEM_SHARED((_NPAD,), jnp.float32),
        pltpu.VMEM_SHARED((_NPAD,), jnp.float32),
    ],
)(_seg_body)


_BN = 1024


def _u_body(x_ref, w1_ref, b1_ref, w2_ref, b2_ref, deg_ref, out_ref):
    a = jnp.dot(x_ref[...], w1_ref[...], preferred_element_type=jnp.float32)
    a = jnp.maximum(a + b1_ref[...][None, :], 0.0)
    h = jnp.dot(a, w2_ref[...], preferred_element_type=jnp.float32)
    u = jnp.sum(h, axis=1) + jnp.sum(b2_ref[...])
    out_ref[...] = u * lax.rsqrt(deg_ref[0, :] + 1.0)


def _u_call(x_p, W1, b1, W2, b2, degs):
    grid = _NPAD // _BN
    return pl.pallas_call(
        _u_body,
        grid=(grid,),
        in_specs=[
            pl.BlockSpec((_BN, _D), lambda i: (i, 0)),
            pl.BlockSpec((_D, _HID), lambda i: (0, 0)),
            pl.BlockSpec((_HID,), lambda i: (0,)),
            pl.BlockSpec((_HID, _HID), lambda i: (0, 0)),
            pl.BlockSpec((_HID,), lambda i: (0,)),
            pl.BlockSpec((_NCORES, _BN), lambda i: (0, i)),
        ],
        out_specs=pl.BlockSpec((_BN,), lambda i: (i,)),
        out_shape=jax.ShapeDtypeStruct((_NPAD,), jnp.float32),
    )(x_p, W1, b1, W2, b2, degs)


def _final_body(rp_ref, t_ref, deg_ref, w3_ref, b3_ref, w4_ref, b4_ref,
                out_ref, acc_ref):
    i = pl.program_id(0)
    r = rp_ref[0, :] + rp_ref[1, :]
    rt = (r + t_ref[...]) * lax.rsqrt(deg_ref[1, :] + 1.0)
    mnf = (rt * (1.0 / _N))[None, :]
    part = jnp.dot(mnf, w3_ref[...], preferred_element_type=jnp.float32)

    @pl.when(i == 0)
    def _():
        acc_ref[...] = jnp.zeros_like(acc_ref)

    acc_ref[...] += part

    @pl.when(i == pl.num_programs(0) - 1)
    def _():
        z = jnp.maximum(acc_ref[...] + b3_ref[...][None, :], 0.0)
        out_ref[...] = (jnp.dot(z, w4_ref[...], preferred_element_type=jnp.float32)
                        + b4_ref[...][None, :])


def _final_call(r_parts, t, degs, W3_p, b3, W4, b4):
    grid = _NPAD // _BN
    return pl.pallas_call(
        _final_body,
        grid=(grid,),
        in_specs=[
            pl.BlockSpec((_NCORES, _BN), lambda i: (0, i)),
            pl.BlockSpec((_BN,), lambda i: (i,)),
            pl.BlockSpec((_NCORES, _BN), lambda i: (0, i)),
            pl.BlockSpec((_BN, _HID), lambda i: (i, 0)),
            pl.BlockSpec((_HID,), lambda i: (0,)),
            pl.BlockSpec((_HID, _NCLS), lambda i: (0, 0)),
            pl.BlockSpec((_NCLS,), lambda i: (0,)),
        ],
        out_specs=pl.BlockSpec((1, _NCLS), lambda i: (0, 0)),
        out_shape=jax.ShapeDtypeStruct((1, _NCLS), jnp.float32),
        scratch_shapes=[pltpu.VMEM((1, _HID), jnp.float32)],
    )(r_parts, t, degs, W3_p, b3, W4, b4)


def kernel(x, edge_index, W1, b1, W2, b2, W3, b3, W4, b4):
    senders = edge_index[0]
    receivers = edge_index[1]
    pad = _EPAD - _E
    pad_idx = _N + (jnp.arange(pad, dtype=jnp.int32) % (_NPAD - _N))
    edges = jnp.stack([
        jnp.concatenate([senders, pad_idx]),
        jnp.concatenate([receivers, pad_idx]),
    ])
    x_p = jnp.pad(x, ((0, _NPAD - _N), (0, 0)))
    W3_p = jnp.pad(W3, ((0, _NPAD - _N), (0, 0)))

    degs = _deg_call(edges)
    t = _u_call(x_p, W1, b1, W2, b2, degs)
    r_parts = _seg_call(edges, t)
    return _final_call(r_parts, t, degs, W3_p, b3, W4, b4)


# speedup vs baseline: 38.2743x; 2.9471x over previous
import functools

import jax
import jax.numpy as jnp
from jax import lax
from jax.experimental import pallas as pl
from jax.experimental.pallas import tpu as pltpu
from jax.experimental.pallas import tpu_sc as plsc

_N = 10000
_D = 128
_HID = 128
_NCLS = 16
_E = 320000

_NTILES = 16
_NCORES = 2
_CHUNK = 128

_NPAD = 10240
_EPAD = 323584
_NODES_PER_TILE = _NPAD // _NTILES
_EDGES_FULL_PER_TILE = _EPAD // _NTILES
_CHUNKS_FULL = _EDGES_FULL_PER_TILE // _CHUNK
_EDGES_HALF_PER_TILE = _EPAD // (_NTILES * _NCORES)
_CHUNKS_HALF = _EDGES_HALF_PER_TILE // _CHUNK

_VL = 16


def _fill(ref, n, value):
    def body(i, _):
        ref[pl.ds(i * _VL, _VL)] = jnp.full((_VL,), value, jnp.float32)
        return 0
    lax.fori_loop(0, n // _VL, body, 0)


def _fill2d(ref, rows, value):
    def body(j, _):
        def inner(k, _):
            ref[j, pl.ds(k * _VL, _VL)] = jnp.full((_VL,), value, jnp.float32)
            return 0
        lax.fori_loop(0, _CHUNK // _VL, inner, 0)
        return 0
    lax.fori_loop(0, rows, body, 0)


def _deg_body(edges_hbm, deg_out, idx_v, ones_v, zeros_v, deg_sh):
    c = lax.axis_index("c")
    s = lax.axis_index("s")
    _fill(ones_v, _EDGES_FULL_PER_TILE, 1.0)
    _fill(zeros_v, _NODES_PER_TILE, 0.0)
    off = s * _NODES_PER_TILE
    pltpu.sync_copy(zeros_v, deg_sh.at[pl.ds(off, _NODES_PER_TILE)])
    plsc.subcore_barrier()

    base = s * _EDGES_FULL_PER_TILE
    pltpu.sync_copy(edges_hbm.at[c, pl.ds(base, _EDGES_FULL_PER_TILE)], idx_v)
    pltpu.sync_copy(ones_v, deg_sh.at[idx_v], add=True)
    plsc.subcore_barrier()
    pltpu.sync_copy(deg_sh.at[pl.ds(off, _NODES_PER_TILE)],
                    deg_out.at[c, pl.ds(off, _NODES_PER_TILE)])


_deg_call = functools.partial(
    pl.kernel,
    out_type=jax.ShapeDtypeStruct((_NCORES, _NPAD), jnp.float32),
    mesh=plsc.VectorSubcoreMesh(core_axis_name="c", subcore_axis_name="s"),
    scratch_types=[
        pltpu.VMEM((_EDGES_FULL_PER_TILE,), jnp.int32),
        pltpu.VMEM((_EDGES_FULL_PER_TILE,), jnp.float32),
        pltpu.VMEM((_NODES_PER_TILE,), jnp.float32),
        pltpu.VMEM_SHARED((_NPAD,), jnp.float32),
    ],
)(_deg_body)


def _seg_body(edges_hbm, t_hbm, r_out, sidx_v, ridx_v, val_v, nb2_v, t_sh, r_sh):
    c = lax.axis_index("c")
    s = lax.axis_index("s")
    off = s * _NODES_PER_TILE
    pltpu.sync_copy(t_hbm.at[pl.ds(off, _NODES_PER_TILE)],
                    t_sh.at[pl.ds(off, _NODES_PER_TILE)])
    _fill(nb2_v, _NODES_PER_TILE, 0.0)
    pltpu.sync_copy(nb2_v, r_sh.at[pl.ds(off, _NODES_PER_TILE)])

    base = (c * _NTILES + s) * _EDGES_HALF_PER_TILE
    pltpu.sync_copy(edges_hbm.at[0, pl.ds(base, _EDGES_HALF_PER_TILE)], sidx_v)
    pltpu.sync_copy(edges_hbm.at[1, pl.ds(base, _EDGES_HALF_PER_TILE)], ridx_v)
    plsc.subcore_barrier()

    pltpu.sync_copy(t_sh.at[sidx_v], val_v)
    pltpu.sync_copy(val_v, r_sh.at[ridx_v], add=True)
    plsc.subcore_barrier()
    pltpu.sync_copy(r_sh.at[pl.ds(off, _NODES_PER_TILE)],
                    r_out.at[c, pl.ds(off, _NODES_PER_TILE)])


_seg_call = functools.partial(
    pl.kernel,
    out_type=jax.ShapeDtypeStruct((_NCORES, _NPAD), jnp.float32),
    mesh=plsc.VectorSubcoreMesh(core_axis_name="c", subcore_axis_name="s"),
    scratch_types=[
        pltpu.VMEM((_EDGES_HALF_PER_TILE,), jnp.int32),
        pltpu.VMEM((_EDGES_HALF_PER_TILE,), jnp.int32),
        pltpu.VMEM((_EDGES_HALF_PER_TILE,), jnp.float32),
        pltpu.VMEM((_NODES_PER_TILE,), jnp.float32),
        pltpu.VMEM_SHARED((_NPAD,), jnp.float32),
        pltpu.VMEM_SHARED((_NPAD,), jnp.float32),
    ],
)(_seg_body)


_BN = 1024


def _u_body(x_ref, w1_ref, b1_ref, w2_ref, b2_ref, deg_ref, out_ref):
    a = jnp.dot(x_ref[...], w1_ref[...], preferred_element_type=jnp.float32)
    a = jnp.maximum(a + b1_ref[...][None, :], 0.0)
    h = jnp.dot(a, w2_ref[...], preferred_element_type=jnp.float32)
    u = jnp.sum(h, axis=1) + jnp.sum(b2_ref[...])
    out_ref[...] = u * lax.rsqrt(deg_ref[0, :] + 1.0)


def _u_call(x_p, W1, b1, W2, b2, degs):
    grid = _NPAD // _BN
    return pl.pallas_call(
        _u_body,
        grid=(grid,),
        in_specs=[
            pl.BlockSpec((_BN, _D), lambda i: (i, 0)),
            pl.BlockSpec((_D, _HID), lambda i: (0, 0)),
            pl.BlockSpec((_HID,), lambda i: (0,)),
            pl.BlockSpec((_HID, _HID), lambda i: (0, 0)),
            pl.BlockSpec((_HID,), lambda i: (0,)),
            pl.BlockSpec((_NCORES, _BN), lambda i: (0, i)),
        ],
        out_specs=pl.BlockSpec((_BN,), lambda i: (i,)),
        out_shape=jax.ShapeDtypeStruct((_NPAD,), jnp.float32),
    )(x_p, W1, b1, W2, b2, degs)


def _final_body(rp_ref, t_ref, deg_ref, w3_ref, b3_ref, w4_ref, b4_ref,
                out_ref, acc_ref):
    i = pl.program_id(0)
    r = rp_ref[0, :] + rp_ref[1, :]
    rt = (r + t_ref[...]) * lax.rsqrt(deg_ref[1, :] + 1.0)
    mnf = (rt * (1.0 / _N))[None, :]
    part = jnp.dot(mnf, w3_ref[...], preferred_element_type=jnp.float32)

    @pl.when(i == 0)
    def _():
        acc_ref[...] = jnp.zeros_like(acc_ref)

    acc_ref[...] += part

    @pl.when(i == pl.num_programs(0) - 1)
    def _():
        z = jnp.maximum(acc_ref[...] + b3_ref[...][None, :], 0.0)
        out_ref[...] = (jnp.dot(z, w4_ref[...], preferred_element_type=jnp.float32)
                        + b4_ref[...][None, :])


def _final_call(r_parts, t, degs, W3_p, b3, W4, b4):
    grid = _NPAD // _BN
    return pl.pallas_call(
        _final_body,
        grid=(grid,),
        in_specs=[
            pl.BlockSpec((_NCORES, _BN), lambda i: (0, i)),
            pl.BlockSpec((_BN,), lambda i: (i,)),
            pl.BlockSpec((_NCORES, _BN), lambda i: (0, i)),
            pl.BlockSpec((_BN, _HID), lambda i: (i, 0)),
            pl.BlockSpec((_HID,), lambda i: (0,)),
            pl.BlockSpec((_HID, _NCLS), lambda i: (0, 0)),
            pl.BlockSpec((_NCLS,), lambda i: (0,)),
        ],
        out_specs=pl.BlockSpec((1, _NCLS), lambda i: (0, 0)),
        out_shape=jax.ShapeDtypeStruct((1, _NCLS), jnp.float32),
        scratch_shapes=[pltpu.VMEM((1, _HID), jnp.float32)],
    )(r_parts, t, degs, W3_p, b3, W4, b4)


def kernel(x, edge_index, W1, b1, W2, b2, W3, b3, W4, b4):
    senders = edge_index[0]
    receivers = edge_index[1]
    pad = _EPAD - _E
    pad_idx = _N + (jnp.arange(pad, dtype=jnp.int32) % (_NPAD - _N))
    edges = jnp.stack([
        jnp.concatenate([senders, pad_idx]),
        jnp.concatenate([receivers, pad_idx]),
    ])
    x_p = jnp.pad(x, ((0, _NPAD - _N), (0, 0)))
    W3_p = jnp.pad(W3, ((0, _NPAD - _N), (0, 0)))

    degs = _deg_call(edges)
    t = _u_call(x_p, W1, b1, W2, b2, degs)
    r_parts = _seg_call(edges, t)
    return _final_call(r_parts, t, degs, W3_p, b3, W4, b4)


# speedup vs baseline: 40.2778x; 1.0523x over previous
import functools

import jax
import jax.numpy as jnp
from jax import lax
from jax.experimental import pallas as pl
from jax.experimental.pallas import tpu as pltpu
from jax.experimental.pallas import tpu_sc as plsc

_N = 10000
_D = 128
_HID = 128
_NCLS = 16
_E = 320000

_NTILES = 16
_NCORES = 2
_CHUNK = 128

_NPAD = 10240
_NODES_PER_TILE = _NPAD // _NTILES
_EDGES_FULL_PER_TILE = _E // _NTILES
_EDGES_HALF_PER_TILE = _E // (_NTILES * _NCORES)

_VL = 16


def _fill(ref, n, value):
    def body(i, _):
        ref[pl.ds(i * _VL, _VL)] = jnp.full((_VL,), value, jnp.float32)
        return 0
    lax.fori_loop(0, n // _VL, body, 0)


def _fill2d(ref, rows, value):
    def body(j, _):
        def inner(k, _):
            ref[j, pl.ds(k * _VL, _VL)] = jnp.full((_VL,), value, jnp.float32)
            return 0
        lax.fori_loop(0, _CHUNK // _VL, inner, 0)
        return 0
    lax.fori_loop(0, rows, body, 0)


def _deg_body(send_hbm, recv_hbm, deg_out, idx_v, ones_v, zeros_v, deg_sh):
    c = lax.axis_index("c")
    s = lax.axis_index("s")
    _fill(ones_v, _EDGES_FULL_PER_TILE, 1.0)
    _fill(zeros_v, _NODES_PER_TILE, 0.0)
    off = s * _NODES_PER_TILE
    pltpu.sync_copy(zeros_v, deg_sh.at[pl.ds(off, _NODES_PER_TILE)])
    plsc.subcore_barrier()

    base = s * _EDGES_FULL_PER_TILE

    @pl.when(c == 0)
    def _():
        pltpu.sync_copy(send_hbm.at[pl.ds(base, _EDGES_FULL_PER_TILE)], idx_v)

    @pl.when(c == 1)
    def _():
        pltpu.sync_copy(recv_hbm.at[pl.ds(base, _EDGES_FULL_PER_TILE)], idx_v)

    pltpu.sync_copy(ones_v, deg_sh.at[idx_v], add=True)
    plsc.subcore_barrier()
    pltpu.sync_copy(deg_sh.at[pl.ds(off, _NODES_PER_TILE)],
                    deg_out.at[c, pl.ds(off, _NODES_PER_TILE)])


_deg_call = functools.partial(
    pl.kernel,
    out_type=jax.ShapeDtypeStruct((_NCORES, _NPAD), jnp.float32),
    mesh=plsc.VectorSubcoreMesh(core_axis_name="c", subcore_axis_name="s"),
    scratch_types=[
        pltpu.VMEM((_EDGES_FULL_PER_TILE,), jnp.int32),
        pltpu.VMEM((_EDGES_FULL_PER_TILE,), jnp.float32),
        pltpu.VMEM((_NODES_PER_TILE,), jnp.float32),
        pltpu.VMEM_SHARED((_NPAD,), jnp.float32),
    ],
)(_deg_body)


def _seg_body(send_hbm, recv_hbm, t_hbm, r_out, sidx_v, ridx_v, val_v, nb2_v, t_sh, r_sh):
    c = lax.axis_index("c")
    s = lax.axis_index("s")
    off = s * _NODES_PER_TILE
    pltpu.sync_copy(t_hbm.at[pl.ds(off, _NODES_PER_TILE)],
                    t_sh.at[pl.ds(off, _NODES_PER_TILE)])
    _fill(nb2_v, _NODES_PER_TILE, 0.0)
    pltpu.sync_copy(nb2_v, r_sh.at[pl.ds(off, _NODES_PER_TILE)])

    base = (c * _NTILES + s) * _EDGES_HALF_PER_TILE
    pltpu.sync_copy(send_hbm.at[pl.ds(base, _EDGES_HALF_PER_TILE)], sidx_v)
    pltpu.sync_copy(recv_hbm.at[pl.ds(base, _EDGES_HALF_PER_TILE)], ridx_v)
    plsc.subcore_barrier()

    pltpu.sync_copy(t_sh.at[sidx_v], val_v)
    pltpu.sync_copy(val_v, r_sh.at[ridx_v], add=True)
    plsc.subcore_barrier()
    pltpu.sync_copy(r_sh.at[pl.ds(off, _NODES_PER_TILE)],
                    r_out.at[c, pl.ds(off, _NODES_PER_TILE)])


_seg_call = functools.partial(
    pl.kernel,
    out_type=jax.ShapeDtypeStruct((_NCORES, _NPAD), jnp.float32),
    mesh=plsc.VectorSubcoreMesh(core_axis_name="c", subcore_axis_name="s"),
    scratch_types=[
        pltpu.VMEM((_EDGES_HALF_PER_TILE,), jnp.int32),
        pltpu.VMEM((_EDGES_HALF_PER_TILE,), jnp.int32),
        pltpu.VMEM((_EDGES_HALF_PER_TILE,), jnp.float32),
        pltpu.VMEM((_NODES_PER_TILE,), jnp.float32),
        pltpu.VMEM_SHARED((_NPAD,), jnp.float32),
        pltpu.VMEM_SHARED((_NPAD,), jnp.float32),
    ],
)(_seg_body)


_BN = 1024


def _u_body(x_ref, w1_ref, b1_ref, w2_ref, b2_ref, deg_ref, out_ref):
    a = jnp.dot(x_ref[...], w1_ref[...], preferred_element_type=jnp.float32)
    a = jnp.maximum(a + b1_ref[...][None, :], 0.0)
    h = jnp.dot(a, w2_ref[...], preferred_element_type=jnp.float32)
    u = jnp.sum(h, axis=1) + jnp.sum(b2_ref[...])
    out_ref[...] = u * lax.rsqrt(deg_ref[0, :] + 1.0)


def _u_call(x_p, W1, b1, W2, b2, degs):
    grid = _NPAD // _BN
    return pl.pallas_call(
        _u_body,
        grid=(grid,),
        in_specs=[
            pl.BlockSpec((_BN, _D), lambda i: (i, 0)),
            pl.BlockSpec((_D, _HID), lambda i: (0, 0)),
            pl.BlockSpec((_HID,), lambda i: (0,)),
            pl.BlockSpec((_HID, _HID), lambda i: (0, 0)),
            pl.BlockSpec((_HID,), lambda i: (0,)),
            pl.BlockSpec((_NCORES, _BN), lambda i: (0, i)),
        ],
        out_specs=pl.BlockSpec((_BN,), lambda i: (i,)),
        out_shape=jax.ShapeDtypeStruct((_NPAD,), jnp.float32),
    )(x_p, W1, b1, W2, b2, degs)


def _final_body(rp_ref, t_ref, deg_ref, w3_ref, b3_ref, w4_ref, b4_ref,
                out_ref, acc_ref):
    i = pl.program_id(0)
    base = i * _BN
    r = rp_ref[0, :] + rp_ref[1, :]
    rt = (r + t_ref[...]) * lax.rsqrt(deg_ref[1, :] + 1.0)
    col = lax.broadcasted_iota(jnp.int32, (_BN,), 0) + base
    mnf = jnp.where(col < _N, rt * (1.0 / _N), 0.0)[None, :]
    row = lax.broadcasted_iota(jnp.int32, (_BN, 1), 0) + base
    w3 = jnp.where(row < _N, w3_ref[...], 0.0)
    part = jnp.dot(mnf, w3, preferred_element_type=jnp.float32)

    @pl.when(i == 0)
    def _():
        acc_ref[...] = jnp.zeros_like(acc_ref)

    acc_ref[...] += part

    @pl.when(i == pl.num_programs(0) - 1)
    def _():
        z = jnp.maximum(acc_ref[...] + b3_ref[...][None, :], 0.0)
        out_ref[...] = (jnp.dot(z, w4_ref[...], preferred_element_type=jnp.float32)
                        + b4_ref[...][None, :])


def _final_call(r_parts, t, degs, W3_p, b3, W4, b4):
    grid = _NPAD // _BN
    return pl.pallas_call(
        _final_body,
        grid=(grid,),
        in_specs=[
            pl.BlockSpec((_NCORES, _BN), lambda i: (0, i)),
            pl.BlockSpec((_BN,), lambda i: (i,)),
            pl.BlockSpec((_NCORES, _BN), lambda i: (0, i)),
            pl.BlockSpec((_BN, _HID), lambda i: (i, 0)),
            pl.BlockSpec((_HID,), lambda i: (0,)),
            pl.BlockSpec((_HID, _NCLS), lambda i: (0, 0)),
            pl.BlockSpec((_NCLS,), lambda i: (0,)),
        ],
        out_specs=pl.BlockSpec((1, _NCLS), lambda i: (0, 0)),
        out_shape=jax.ShapeDtypeStruct((1, _NCLS), jnp.float32),
        scratch_shapes=[pltpu.VMEM((1, _HID), jnp.float32)],
    )(r_parts, t, degs, W3_p, b3, W4, b4)


def kernel(x, edge_index, W1, b1, W2, b2, W3, b3, W4, b4):
    senders = edge_index[0]
    receivers = edge_index[1]
    degs = _deg_call(senders, receivers)
    t = _u_call(x, W1, b1, W2, b2, degs)
    r_parts = _seg_call(senders, receivers, t)
    return _final_call(r_parts, t, degs, W3, b3, W4, b4)
